# baseline (speedup 1.0000x reference)
import jax
import jax.numpy as jnp
from jax.experimental import pallas as pl

_BLOCK = 2048


def _mlp_body(x_ref, w0a_ref, w0b_ref, b0_ref, w1t_ref, b1_ref, o_ref):
    gus = x_ref[0]
    gis = x_ref[1]
    h = jnp.dot(gus, w0a_ref[...], preferred_element_type=jnp.float32)
    h = h + jnp.dot(gis, w0b_ref[...], preferred_element_type=jnp.float32)
    h = jnp.maximum(h + b0_ref[...], 0.0)
    o_ref[...] = jnp.sum(h * w1t_ref[...], axis=1, keepdims=True) + b1_ref[...]


def kernel(inputs, W0, b0, W1, b1):
    _, B, K = inputs.shape
    H = W0.shape[1]
    w0a = W0[:K]
    w0b = W0[K:]
    b0r = b0.reshape(1, H)
    w1t = W1.reshape(1, H)
    b1r = b1.reshape(1, 1)
    return pl.pallas_call(
        _mlp_body,
        grid=(B // _BLOCK,),
        in_specs=[
            pl.BlockSpec((2, _BLOCK, K), lambda i: (0, i, 0)),
            pl.BlockSpec((K, H), lambda i: (0, 0)),
            pl.BlockSpec((K, H), lambda i: (0, 0)),
            pl.BlockSpec((1, H), lambda i: (0, 0)),
            pl.BlockSpec((1, H), lambda i: (0, 0)),
            pl.BlockSpec((1, 1), lambda i: (0, 0)),
        ],
        out_specs=pl.BlockSpec((_BLOCK, 1), lambda i: (i, 0)),
        out_shape=jax.ShapeDtypeStruct((B, 1), jnp.float32),
    )(inputs, w0a, w0b, b0r, w1t, b1r)
